# baseline (speedup 1.0000x reference)
import functools

import jax
import jax.numpy as jnp
from jax import lax
from jax.experimental import pallas as pl
from jax.experimental.pallas import tpu as pltpu
from jax.experimental.pallas import tpu_sc as plsc

HIDDEN = 4096
NUM_EXPERTS = 64
TOP_K = 8
N_GROUPS = 8
EPG = NUM_EXPERTS // N_GROUPS
TOPK_GROUPS = 4
ROUTED_SCALING_FACTOR = 2.5

NC = 2
NS = 16
NW = NC * NS
L = 16
CB = 128


def _score_block(x_ref, w_ref, b_ref, sb_ref):
    x = x_ref[...]
    w = w_ref[...]
    s = jax.nn.sigmoid(jnp.dot(x, w, preferred_element_type=jnp.float32))
    sb_ref[...] = s + b_ref[...]


def _scores_tc(x_TD, kernel_DE, bias_E, tb=512):
    t = x_TD.shape[0]
    bias_2d = jnp.reshape(bias_E, (1, NUM_EXPERTS)).astype(jnp.float32)
    return pl.pallas_call(
        _score_block,
        grid=(t // tb,),
        in_specs=[
            pl.BlockSpec((tb, HIDDEN), lambda i: (i, 0)),
            pl.BlockSpec((HIDDEN, NUM_EXPERTS), lambda i: (0, 0)),
            pl.BlockSpec((1, NUM_EXPERTS), lambda i: (0, 0)),
        ],
        out_specs=pl.BlockSpec((tb, NUM_EXPERTS), lambda i: (i, 0)),
        out_shape=jax.ShapeDtypeStruct((t, NUM_EXPERTS), jnp.float32),
    )(x_TD, kernel_DE, bias_2d)


def _sc_router_body(sb_hbm, b2_hbm, wout_hbm, iout_hbm,
                    sb_chunk, ms_ref, bias_v, wv, iv):
    wid = lax.axis_index("s") * NC + lax.axis_index("c")
    t_total = sb_hbm.shape[0]
    tw = t_total // NW
    nst = tw // CB
    nsb = CB // L
    pltpu.sync_copy(b2_hbm, bias_v)
    iota = lax.iota(jnp.int32, L)
    zero16 = jnp.zeros((L,), jnp.int32)
    neg = jnp.full((L,), -1e30, jnp.float32)
    zero = jnp.full((L,), 0.0, jnp.float32)
    one = jnp.full((L,), 1, jnp.int32)
    esplat = [jnp.full((L,), e, jnp.int32) for e in range(NUM_EXPERTS)]

    def stage_body(st, carry0):
        t0 = wid * tw + st * CB
        pltpu.sync_copy(sb_hbm.at[pl.ds(t0, CB)], sb_chunk)

        def sub_body(i, carry):
            row = iota + i * L

            gs = []
            for g in range(N_GROUPS):
                m1 = m2 = None
                for o in range(EPG):
                    e = EPG * g + o
                    sb = plsc.load_gather(sb_chunk, [row, esplat[e]])
                    ms_ref[e] = sb
                    if o == 0:
                        m1, m2 = sb, neg
                    else:
                        m2 = jnp.maximum(m2, jnp.minimum(sb, m1))
                        m1 = jnp.maximum(m1, sb)
                gs.append(m1 + m2)

            rank = [jnp.zeros((L,), jnp.int32) for _ in range(N_GROUPS)]
            for g in range(N_GROUPS):
                for h in range(g + 1, N_GROUPS):
                    c = (gs[h] > gs[g]).astype(jnp.int32)
                    rank[g] = rank[g] + c
                    rank[h] = rank[h] + (one - c)
            sel = [rank[g] < TOPK_GROUPS for g in range(N_GROUPS)]

            for e in range(NUM_EXPERTS):
                ms_ref[e] = jnp.where(sel[e // EPG], ms_ref[e], zero)

            wcols, icols = [], []
            for j in range(TOP_K):
                vcur = [ms_ref[e] for e in range(NUM_EXPERTS)]
                icur = list(esplat)
                n = NUM_EXPERTS
                while n > 1:
                    nv, ni = [], []
                    for k in range(0, n, 2):
                        c = vcur[k + 1] > vcur[k]
                        nv.append(jnp.where(c, vcur[k + 1], vcur[k]))
                        ni.append(jnp.where(c, icur[k + 1], icur[k]))
                    vcur, icur = nv, ni
                    n //= 2
                m, mi = vcur[0], icur[0]
                be = plsc.load_gather(bias_v, [zero16, mi])
                wcols.append(m - be)
                icols.append(mi)
                if j + 1 < TOP_K:
                    plsc.store_scatter(ms_ref, [mi, iota], neg)

            den = wcols[0]
            for j in range(1, TOP_K):
                den = den + wcols[j]
            den = den + 1e-20
            for j in range(TOP_K):
                plsc.store_scatter(
                    wv, [row, esplat[j]],
                    wcols[j] / den * ROUTED_SCALING_FACTOR)
                plsc.store_scatter(iv, [row, esplat[j]], icols[j])
            return carry

        lax.fori_loop(0, nsb, sub_body, 0)

        pltpu.sync_copy(wv, wout_hbm.at[pl.ds(t0, CB)])
        pltpu.sync_copy(iv, iout_hbm.at[pl.ds(t0, CB)])
        return carry0

    lax.fori_loop(0, nst, stage_body, 0)


def _make_sc_router(t):
    mesh = plsc.VectorSubcoreMesh(core_axis_name="c", subcore_axis_name="s")
    return pl.kernel(
        _sc_router_body,
        out_type=[
            jax.ShapeDtypeStruct((t, TOP_K), jnp.float32),
            jax.ShapeDtypeStruct((t, TOP_K), jnp.int32),
        ],
        mesh=mesh,
        compiler_params=pltpu.CompilerParams(needs_layout_passes=False),
        scratch_types=[
            pltpu.VMEM((CB, NUM_EXPERTS), jnp.float32),
            pltpu.VMEM((NUM_EXPERTS, L), jnp.float32),
            pltpu.VMEM((1, NUM_EXPERTS), jnp.float32),
            pltpu.VMEM((CB, TOP_K), jnp.float32),
            pltpu.VMEM((CB, TOP_K), jnp.int32),
        ],
    )


@functools.partial(jax.jit, static_argnames=())
def kernel(x_TD, kernel_DE, bias_E):
    x_TD = jnp.asarray(x_TD, jnp.float32)
    t = x_TD.shape[0]
    bias_2d = jnp.reshape(bias_E, (1, NUM_EXPERTS)).astype(jnp.float32)
    nchunk = 4
    tc = t // nchunk
    router = _make_sc_router(tc)
    wparts, iparts = [], []
    for c in range(nchunk):
        xc = lax.slice_in_dim(x_TD, c * tc, (c + 1) * tc, axis=0)
        sb_c = _scores_tc(xc, kernel_DE, bias_E)
        w_c, i_c = router(sb_c, bias_2d)
        wparts.append(w_c)
        iparts.append(i_c)
    return (jnp.concatenate(wparts, axis=0),
            jnp.concatenate(iparts, axis=0))


# speedup vs baseline: 1.7827x; 1.7827x over previous
import functools

import jax
import jax.numpy as jnp
from jax import lax
from jax.experimental import pallas as pl
from jax.experimental.pallas import tpu as pltpu
from jax.experimental.pallas import tpu_sc as plsc

HIDDEN = 4096
NUM_EXPERTS = 64
TOP_K = 8
N_GROUPS = 8
EPG = NUM_EXPERTS // N_GROUPS
TOPK_GROUPS = 4
ROUTED_SCALING_FACTOR = 2.5

NC = 2
NS = 16
NW = NC * NS
L = 16
CB = 128


def _score_block(x_ref, w_ref, b_ref, sb_ref):
    x = x_ref[...]
    w = w_ref[...]
    s = jax.nn.sigmoid(jnp.dot(x, w, preferred_element_type=jnp.float32))
    sb_ref[...] = s + b_ref[...]


def _scores_tc(x_TD, kernel_DE, bias_E, tb=512):
    t = x_TD.shape[0]
    bias_2d = jnp.reshape(bias_E, (1, NUM_EXPERTS)).astype(jnp.float32)
    return pl.pallas_call(
        _score_block,
        grid=(t // tb,),
        in_specs=[
            pl.BlockSpec((tb, HIDDEN), lambda i: (i, 0)),
            pl.BlockSpec((HIDDEN, NUM_EXPERTS), lambda i: (0, 0)),
            pl.BlockSpec((1, NUM_EXPERTS), lambda i: (0, 0)),
        ],
        out_specs=pl.BlockSpec((tb, NUM_EXPERTS), lambda i: (i, 0)),
        out_shape=jax.ShapeDtypeStruct((t, NUM_EXPERTS), jnp.float32),
    )(x_TD, kernel_DE, bias_2d)


def _sc_router_body(sb_hbm, b2_hbm, wout_hbm, iout_hbm,
                    sb_chunk, ms_ref, bias_v, wv, iv):
    wid = lax.axis_index("s") * NC + lax.axis_index("c")
    t_total = sb_hbm.shape[0]
    tw = t_total // NW
    nst = tw // CB
    nsb = CB // L
    pltpu.sync_copy(b2_hbm, bias_v)
    iota = lax.iota(jnp.int32, L)
    zero16 = jnp.zeros((L,), jnp.int32)
    neg = jnp.full((L,), -1e30, jnp.float32)
    zero = jnp.full((L,), 0.0, jnp.float32)
    one = jnp.full((L,), 1, jnp.int32)
    esplat = [jnp.full((L,), e, jnp.int32) for e in range(NUM_EXPERTS)]

    def stage_body(st, carry0):
        t0 = wid * tw + st * CB
        pltpu.sync_copy(sb_hbm.at[pl.ds(t0, CB)], sb_chunk)

        def sub_body(i, carry):
            row = iota + i * L

            gs = []
            for g in range(N_GROUPS):
                m1 = m2 = None
                for o in range(EPG):
                    e = EPG * g + o
                    sb = plsc.load_gather(sb_chunk, [row, esplat[e]])
                    ms_ref[e] = sb
                    if o == 0:
                        m1, m2 = sb, neg
                    else:
                        m2 = jnp.maximum(m2, jnp.minimum(sb, m1))
                        m1 = jnp.maximum(m1, sb)
                gs.append(m1 + m2)

            rank = [jnp.zeros((L,), jnp.int32) for _ in range(N_GROUPS)]
            for g in range(N_GROUPS):
                for h in range(g + 1, N_GROUPS):
                    c = (gs[h] > gs[g]).astype(jnp.int32)
                    rank[g] = rank[g] + c
                    rank[h] = rank[h] + (one - c)
            sel = [rank[g] < TOPK_GROUPS for g in range(N_GROUPS)]

            for e in range(NUM_EXPERTS):
                ms_ref[e] = jnp.where(sel[e // EPG], ms_ref[e], zero)

            wcols, icols = [], []
            for j in range(TOP_K):
                vcur = [ms_ref[e] for e in range(NUM_EXPERTS)]
                icur = list(esplat)
                n = NUM_EXPERTS
                while n > 1:
                    nv, ni = [], []
                    for k in range(0, n, 2):
                        c = vcur[k + 1] > vcur[k]
                        nv.append(jnp.where(c, vcur[k + 1], vcur[k]))
                        ni.append(jnp.where(c, icur[k + 1], icur[k]))
                    vcur, icur = nv, ni
                    n //= 2
                m, mi = vcur[0], icur[0]
                be = plsc.load_gather(bias_v, [zero16, mi])
                wcols.append(m - be)
                icols.append(mi)
                if j + 1 < TOP_K:
                    plsc.store_scatter(ms_ref, [mi, iota], neg)

            den = wcols[0]
            for j in range(1, TOP_K):
                den = den + wcols[j]
            den = den + 1e-20
            for j in range(TOP_K):
                plsc.store_scatter(
                    wv, [row, esplat[j]],
                    wcols[j] / den * ROUTED_SCALING_FACTOR)
                plsc.store_scatter(iv, [row, esplat[j]], icols[j])
            return carry

        lax.fori_loop(0, nsb, sub_body, 0)

        pltpu.sync_copy(wv, wout_hbm.at[pl.ds(t0, CB)])
        pltpu.sync_copy(iv, iout_hbm.at[pl.ds(t0, CB)])
        return carry0

    lax.fori_loop(0, nst, stage_body, 0)


def _make_sc_router(t):
    mesh = plsc.VectorSubcoreMesh(core_axis_name="c", subcore_axis_name="s")
    return pl.kernel(
        _sc_router_body,
        out_type=[
            jax.ShapeDtypeStruct((t, TOP_K), jnp.float32),
            jax.ShapeDtypeStruct((t, TOP_K), jnp.int32),
        ],
        mesh=mesh,
        compiler_params=pltpu.CompilerParams(needs_layout_passes=False),
        scratch_types=[
            pltpu.VMEM((CB, NUM_EXPERTS), jnp.float32),
            pltpu.VMEM((NUM_EXPERTS, L), jnp.float32),
            pltpu.VMEM((1, NUM_EXPERTS), jnp.float32),
            pltpu.VMEM((CB, TOP_K), jnp.float32),
            pltpu.VMEM((CB, TOP_K), jnp.int32),
        ],
    )


@functools.partial(jax.jit, static_argnames=())
def kernel(x_TD, kernel_DE, bias_E):
    x_TD = jnp.asarray(x_TD, jnp.float32)
    t = x_TD.shape[0]
    bias_2d = jnp.reshape(bias_E, (1, NUM_EXPERTS)).astype(jnp.float32)
    sb_TE = _scores_tc(x_TD, kernel_DE, bias_E)
    router = _make_sc_router(t)
    weights, indices = router(sb_TE, bias_2d)
    return weights, indices


# speedup vs baseline: 1.9990x; 1.1213x over previous
import functools

import jax
import jax.numpy as jnp
from jax import lax
from jax.experimental import pallas as pl
from jax.experimental.pallas import tpu as pltpu
from jax.experimental.pallas import tpu_sc as plsc

HIDDEN = 4096
NUM_EXPERTS = 64
TOP_K = 8
N_GROUPS = 8
EPG = NUM_EXPERTS // N_GROUPS
TOPK_GROUPS = 4
ROUTED_SCALING_FACTOR = 2.5

NC = 2
NS = 16
NW = NC * NS
L = 16
CB = 128


def _score_block(x_ref, w_ref, b_ref, sb_ref):
    x = x_ref[...]
    w = w_ref[...]
    s = jax.nn.sigmoid(jnp.dot(x, w, preferred_element_type=jnp.float32))
    sb_ref[...] = s + b_ref[...]


def _scores_tc(x_TD, kernel_DE, bias_E, tb=512):
    t = x_TD.shape[0]
    bias_2d = jnp.reshape(bias_E, (1, NUM_EXPERTS)).astype(jnp.float32)
    return pl.pallas_call(
        _score_block,
        grid=(t // tb,),
        in_specs=[
            pl.BlockSpec((tb, HIDDEN), lambda i: (i, 0)),
            pl.BlockSpec((HIDDEN, NUM_EXPERTS), lambda i: (0, 0)),
            pl.BlockSpec((1, NUM_EXPERTS), lambda i: (0, 0)),
        ],
        out_specs=pl.BlockSpec((tb, NUM_EXPERTS), lambda i: (i, 0)),
        out_shape=jax.ShapeDtypeStruct((t, NUM_EXPERTS), jnp.float32),
    )(x_TD, kernel_DE, bias_2d)


def _sc_router_body(sb_hbm, b2_hbm, wout_hbm, iout_hbm,
                    sb_chunk, ms_ref, bias_v, wv, iv):
    wid = lax.axis_index("s") * NC + lax.axis_index("c")
    t_total = sb_hbm.shape[0]
    tw = t_total // NW
    nst = tw // CB
    nsb = CB // L
    pltpu.sync_copy(b2_hbm, bias_v)
    iota = lax.iota(jnp.int32, L)
    zero16 = jnp.zeros((L,), jnp.int32)
    neg = jnp.full((L,), -1e30, jnp.float32)
    zero = jnp.full((L,), 0.0, jnp.float32)
    one = jnp.full((L,), 1, jnp.int32)
    esplat = [jnp.full((L,), e, jnp.int32) for e in range(NUM_EXPERTS)]

    def stage_body(st, carry0):
        t0 = wid * tw + st * CB
        pltpu.sync_copy(sb_hbm.at[pl.ds(t0, CB)], sb_chunk)

        def _merge(va, ia, vb, ib):
            c = vb > va
            return jnp.where(c, vb, va), jnp.where(c, ib, ia)

        def sub_body(i, carry):
            row = iota + i * L

            vals = [None] * NUM_EXPERTS
            gs = []
            for g in range(N_GROUPS):
                m1 = m2 = None
                for o in range(EPG):
                    e = EPG * g + o
                    sb = plsc.load_gather(sb_chunk, [row, esplat[e]])
                    vals[e] = sb
                    if o == 0:
                        m1, m2 = sb, neg
                    else:
                        m2 = jnp.maximum(m2, jnp.minimum(sb, m1))
                        m1 = jnp.maximum(m1, sb)
                gs.append(m1 + m2)

            rank = [jnp.zeros((L,), jnp.int32) for _ in range(N_GROUPS)]
            for g in range(N_GROUPS):
                for h in range(g + 1, N_GROUPS):
                    c = (gs[h] > gs[g]).astype(jnp.int32)
                    rank[g] = rank[g] + c
                    rank[h] = rank[h] + (one - c)
            sel = [rank[g] < TOPK_GROUPS for g in range(N_GROUPS)]

            for e in range(NUM_EXPERTS):
                vals[e] = jnp.where(sel[e // EPG], vals[e], zero)
                ms_ref[e] = vals[e]

            gm_v, gm_i = [], []
            for g in range(N_GROUPS):
                vcur = vals[EPG * g:EPG * (g + 1)]
                icur = esplat[EPG * g:EPG * (g + 1)]
                while len(vcur) > 1:
                    nv, ni = [], []
                    for k in range(0, len(vcur), 2):
                        v, ix = _merge(vcur[k], icur[k],
                                       vcur[k + 1], icur[k + 1])
                        nv.append(v)
                        ni.append(ix)
                    vcur, icur = nv, ni
                gm_v.append(vcur[0])
                gm_i.append(icur[0])
            gsplat = [jnp.full((L,), g, jnp.int32) for g in range(N_GROUPS)]

            wcols, icols = [], []
            for j in range(TOP_K):
                vcur = list(gm_v)
                icur = list(gm_i)
                gcur = list(gsplat)
                while len(vcur) > 1:
                    nv, ni, ng = [], [], []
                    for k in range(0, len(vcur), 2):
                        c = vcur[k + 1] > vcur[k]
                        nv.append(jnp.where(c, vcur[k + 1], vcur[k]))
                        ni.append(jnp.where(c, icur[k + 1], icur[k]))
                        ng.append(jnp.where(c, gcur[k + 1], gcur[k]))
                    vcur, icur, gcur = nv, ni, ng
                m, mi, gw = vcur[0], icur[0], gcur[0]
                be = plsc.load_gather(bias_v, [zero16, mi])
                wcols.append(m - be)
                icols.append(mi)
                if j + 1 < TOP_K:
                    plsc.store_scatter(ms_ref, [mi, iota], neg)
                    rowbase = gw * EPG
                    vcur, icur = None, None
                    for o in range(EPG):
                        ro = rowbase + o
                        vo = plsc.load_gather(ms_ref, [ro, iota])
                        if o == 0:
                            vcur, icur = vo, ro
                        else:
                            vcur, icur = _merge(vcur, icur, vo, ro)
                    for g in range(N_GROUPS):
                        hit = gw == gsplat[g]
                        gm_v[g] = jnp.where(hit, vcur, gm_v[g])
                        gm_i[g] = jnp.where(hit, icur, gm_i[g])

            den = wcols[0]
            for j in range(1, TOP_K):
                den = den + wcols[j]
            den = den + 1e-20
            for j in range(TOP_K):
                plsc.store_scatter(
                    wv, [row, esplat[j]],
                    wcols[j] / den * ROUTED_SCALING_FACTOR)
                plsc.store_scatter(iv, [row, esplat[j]], icols[j])
            return carry

        lax.fori_loop(0, nsb, sub_body, 0)

        pltpu.sync_copy(wv, wout_hbm.at[pl.ds(t0, CB)])
        pltpu.sync_copy(iv, iout_hbm.at[pl.ds(t0, CB)])
        return carry0

    lax.fori_loop(0, nst, stage_body, 0)


def _make_sc_router(t):
    mesh = plsc.VectorSubcoreMesh(core_axis_name="c", subcore_axis_name="s")
    return pl.kernel(
        _sc_router_body,
        out_type=[
            jax.ShapeDtypeStruct((t, TOP_K), jnp.float32),
            jax.ShapeDtypeStruct((t, TOP_K), jnp.int32),
        ],
        mesh=mesh,
        compiler_params=pltpu.CompilerParams(needs_layout_passes=False),
        scratch_types=[
            pltpu.VMEM((CB, NUM_EXPERTS), jnp.float32),
            pltpu.VMEM((NUM_EXPERTS, L), jnp.float32),
            pltpu.VMEM((1, NUM_EXPERTS), jnp.float32),
            pltpu.VMEM((CB, TOP_K), jnp.float32),
            pltpu.VMEM((CB, TOP_K), jnp.int32),
        ],
    )


@functools.partial(jax.jit, static_argnames=())
def kernel(x_TD, kernel_DE, bias_E):
    x_TD = jnp.asarray(x_TD, jnp.float32)
    t = x_TD.shape[0]
    bias_2d = jnp.reshape(bias_E, (1, NUM_EXPERTS)).astype(jnp.float32)
    sb_TE = _scores_tc(x_TD, kernel_DE, bias_E)
    router = _make_sc_router(t)
    weights, indices = router(sb_TE, bias_2d)
    return weights, indices
